# baseline (speedup 1.0000x reference)
import math

import jax
import jax.numpy as jnp
from jax import lax
from jax.experimental import pallas as pl
from jax.experimental.pallas import tpu as pltpu

N_SAMPLE = 20
EMB = 512
HEADS = 4
DH = EMB // HEADS
BATCH = 8
NPTS = 16384
TTOK = 64
NVIEW = 4
KPV = N_SAMPLE // NVIEW
LTOT = N_SAMPLE + TTOK


def _fused_body(xyz_ref, mask_ref, pf_ref, t_ref, wq_ref, bq_ref, wk_ref,
                bk_ref, wv_ref, bv_ref, wo_ref, bo_ref, out_ref,
                x_scr, blk_scr, idx_scr, sem):
    i = pl.program_id(0)
    p = lax.rem(i, 2)
    q = 1 - p
    col16 = lax.broadcasted_iota(jnp.int32, (NVIEW, 16), 1)
    row16 = lax.broadcasted_iota(jnp.int32, (NVIEW, 16), 0)
    inf = jnp.float32(jnp.inf)

    @pl.when(i > 0)
    def _process():
        for r in range(N_SAMPLE):
            pltpu.make_async_copy(
                pf_ref.at[0, :, pl.ds(0, 128)],
                blk_scr.at[q, :, pl.ds(r * 128, 128)],
                sem,
            ).wait()
        arr = idx_scr[q]
        lane128 = lax.broadcasted_iota(jnp.int32, (EMB, 128), 1)
        terms = []
        for r in range(N_SAMPLE):
            v_, k_ = r // KPV, r % KPV
            col = jnp.sum(jnp.where((row16 == v_) & (col16 == k_),
                                    arr, 0)) % 128
            blk = blk_scr[q, :, r * 128:(r + 1) * 128]
            rolled = pltpu.roll(blk, jnp.remainder(r - col, 128), axis=1)
            terms.append(jnp.where(lane128 == r, rolled, 0.0))
        while len(terms) > 1:
            terms = [terms[j] + terms[j + 1] if j + 1 < len(terms)
                     else terms[j] for j in range(0, len(terms), 2)]
        S = terms[0]
        eye = (lax.broadcasted_iota(jnp.int32, (EMB, EMB), 0)
               == lax.broadcasted_iota(jnp.int32, (EMB, EMB), 1)
               ).astype(jnp.float32)
        St = lax.dot_general(S, eye, (((0,), (0,)), ((), ())),
                             preferred_element_type=jnp.float32)
        x_scr[0:N_SAMPLE, :] = St[0:N_SAMPLE, :]
        x_scr[N_SAMPLE:LTOT, :] = t_ref[0]
        x = x_scr[...]
        qp = jnp.dot(x, wq_ref[...], preferred_element_type=jnp.float32) + bq_ref[...]
        kp = jnp.dot(x, wk_ref[...], preferred_element_type=jnp.float32) + bk_ref[...]
        vp = jnp.dot(x, wv_ref[...], preferred_element_type=jnp.float32) + bv_ref[...]
        scale = 1.0 / math.sqrt(DH)
        o_heads = []
        for h in range(HEADS):
            c0 = h * DH
            qh = qp[:, c0:c0 + DH]
            kh = kp[:, c0:c0 + DH]
            vh = vp[:, c0:c0 + DH]
            s = lax.dot_general(qh, kh, (((1,), (1,)), ((), ())),
                                preferred_element_type=jnp.float32) * scale
            mx = jnp.max(s, axis=1, keepdims=True)
            e = jnp.exp(s - mx)
            a = e / jnp.sum(e, axis=1, keepdims=True)
            o_heads.append(jnp.dot(a, vh, preferred_element_type=jnp.float32))
        o = jnp.concatenate(o_heads, axis=1)
        out = jnp.dot(o, wo_ref[...], preferred_element_type=jnp.float32) + bo_ref[...]
        out_ref[0] = out

    @pl.when(i < BATCH)
    def _compute():
        x3 = xyz_ref[0]
        m = mask_ref[0]
        cnt = jnp.clip(jnp.sum(m, axis=1), 1.0, None)
        dist2 = jnp.zeros((NVIEW, NPTS), jnp.float32)
        for d in range(3):
            xd = x3[d:d + 1, :]
            cd = jnp.sum(m * xd, axis=1) / cnt
            t = xd - cd[:, None]
            dist2 = dist2 + t * t
        lane = lax.broadcasted_iota(jnp.int32, (NVIEW, NPTS), 1)
        arr = jnp.zeros((NVIEW, 16), jnp.int32)
        d2 = dist2
        for k in range(KPV):
            mn = jnp.min(d2, axis=1, keepdims=True)
            cand = jnp.where(d2 == mn, lane, NPTS)
            mi = jnp.min(cand, axis=1, keepdims=True)
            arr = jnp.where(col16 == k, mi, arr)
            d2 = jnp.where(lane == mi, inf, d2)
        idx_scr[p] = arr
        for v in range(NVIEW):
            for k in range(KPV):
                gF = jnp.sum(jnp.where((row16 == v) & (col16 == k), arr, 0))
                r = v * KPV + k
                pltpu.make_async_copy(
                    pf_ref.at[i, :, pl.ds(pl.multiple_of((gF // 128) * 128, 128), 128)],
                    blk_scr.at[p, :, pl.ds(r * 128, 128)],
                    sem,
                ).start()


def _fused(xyz, masks, pf, t_feat, Wq, bq, Wk, bk, Wv, bv, Wo, bo,
           *, interpret=False):
    last = BATCH - 1
    return pl.pallas_call(
        _fused_body,
        grid=(BATCH + 1,),
        in_specs=[
            pl.BlockSpec((1, 3, NPTS), lambda i: (jnp.minimum(i, last), 0, 0)),
            pl.BlockSpec((1, NVIEW, NPTS),
                         lambda i: (jnp.minimum(i, last), 0, 0)),
            pl.BlockSpec(memory_space=pl.ANY),
            pl.BlockSpec((1, TTOK, EMB),
                         lambda i: (jnp.maximum(i - 1, 0), 0, 0)),
            pl.BlockSpec((EMB, EMB), lambda i: (0, 0)),
            pl.BlockSpec((1, EMB), lambda i: (0, 0)),
            pl.BlockSpec((EMB, EMB), lambda i: (0, 0)),
            pl.BlockSpec((1, EMB), lambda i: (0, 0)),
            pl.BlockSpec((EMB, EMB), lambda i: (0, 0)),
            pl.BlockSpec((1, EMB), lambda i: (0, 0)),
            pl.BlockSpec((EMB, EMB), lambda i: (0, 0)),
            pl.BlockSpec((1, EMB), lambda i: (0, 0)),
        ],
        out_specs=pl.BlockSpec((1, LTOT, EMB),
                               lambda i: (jnp.maximum(i - 1, 0), 0, 0)),
        out_shape=jax.ShapeDtypeStruct((BATCH, LTOT, EMB), jnp.float32),
        scratch_shapes=[
            pltpu.VMEM((LTOT, EMB), jnp.float32),
            pltpu.VMEM((2, EMB, N_SAMPLE * 128), jnp.float32),
            pltpu.VMEM((2, NVIEW, 16), jnp.int32),
            pltpu.SemaphoreType.DMA,
        ],
        interpret=interpret,
    )(xyz, masks, pf, t_feat, Wq, bq, Wk, bk, Wv, bv, Wo, bo)


def kernel(point_features, point_masks, t_feat, t_mask, xyz, Wq, bq, Wk, bk,
           Wv, bv, Wo, bo):
    output = _fused(xyz, point_masks, point_features, t_feat,
                    Wq, bq.reshape(1, EMB), Wk, bk.reshape(1, EMB),
                    Wv, bv.reshape(1, EMB), Wo, bo.reshape(1, EMB))
    combined_mask = jnp.concatenate(
        [jnp.ones((BATCH, N_SAMPLE), dtype=bool), t_mask], axis=1)
    return (output, combined_mask)


# speedup vs baseline: 1.3549x; 1.3549x over previous
import math

import jax
import jax.numpy as jnp
from jax import lax
from jax.experimental import pallas as pl
from jax.experimental.pallas import tpu as pltpu

N_SAMPLE = 20
EMB = 512
HEADS = 4
DH = EMB // HEADS
BATCH = 8
NPTS = 16384
TTOK = 64
NVIEW = 4
KPV = N_SAMPLE // NVIEW
LTOT = N_SAMPLE + TTOK


def _fused_body(xyz_ref, mask_ref, pf_ref, t_ref, wq_ref, bq_ref, wk_ref,
                bk_ref, wv_ref, bv_ref, wo_ref, bo_ref, out_ref,
                x_scr, blk_scr, idx_scr, sem):
    i = pl.program_id(0)
    p = lax.rem(i, 2)
    q = 1 - p
    col16 = lax.broadcasted_iota(jnp.int32, (NVIEW, 16), 1)
    row16 = lax.broadcasted_iota(jnp.int32, (NVIEW, 16), 0)
    inf = jnp.float32(jnp.inf)

    @pl.when(i < BATCH)
    def _compute():
        x3 = xyz_ref[0]
        m = mask_ref[0]
        cnt = jnp.clip(jnp.sum(m, axis=1), 1.0, None)
        dist2 = jnp.zeros((NVIEW, NPTS), jnp.float32)
        for d in range(3):
            xd = x3[d:d + 1, :]
            cd = jnp.sum(m * xd, axis=1) / cnt
            t = xd - cd[:, None]
            dist2 = dist2 + t * t
        lane = lax.broadcasted_iota(jnp.int32, (NVIEW, NPTS), 1)
        arr = jnp.zeros((NVIEW, 16), jnp.int32)
        d2 = dist2
        for k in range(KPV):
            mn = jnp.min(d2, axis=1, keepdims=True)
            cand = jnp.where(d2 == mn, lane, NPTS)
            mi = jnp.min(cand, axis=1, keepdims=True)
            arr = jnp.where(col16 == k, mi, arr)
            d2 = jnp.where(lane == mi, inf, d2)
        idx_scr[p] = arr
        for v in range(NVIEW):
            for k in range(KPV):
                gF = jnp.sum(jnp.where((row16 == v) & (col16 == k), arr, 0))
                r = v * KPV + k
                pltpu.make_async_copy(
                    pf_ref.at[i, :, pl.ds(pl.multiple_of((gF // 128) * 128, 128), 128)],
                    blk_scr.at[p, :, pl.ds(r * 128, 128)],
                    sem.at[p],
                ).start()

    @pl.when(i > 0)
    def _process():
        for r in range(N_SAMPLE):
            pltpu.make_async_copy(
                pf_ref.at[0, :, pl.ds(0, 128)],
                blk_scr.at[q, :, pl.ds(r * 128, 128)],
                sem.at[q],
            ).wait()
        arr = idx_scr[q]
        lane128 = lax.broadcasted_iota(jnp.int32, (EMB, 128), 1)
        terms = []
        for r in range(N_SAMPLE):
            v_, k_ = r // KPV, r % KPV
            col = jnp.sum(jnp.where((row16 == v_) & (col16 == k_),
                                    arr, 0)) % 128
            blk = blk_scr[q, :, r * 128:(r + 1) * 128]
            rolled = pltpu.roll(blk, jnp.remainder(r - col, 128), axis=1)
            terms.append(jnp.where(lane128 == r, rolled, 0.0))
        while len(terms) > 1:
            terms = [terms[j] + terms[j + 1] if j + 1 < len(terms)
                     else terms[j] for j in range(0, len(terms), 2)]
        S = terms[0]
        eye = (lax.broadcasted_iota(jnp.int32, (EMB, EMB), 0)
               == lax.broadcasted_iota(jnp.int32, (EMB, EMB), 1)
               ).astype(jnp.float32)
        St = lax.dot_general(S, eye, (((0,), (0,)), ((), ())),
                             preferred_element_type=jnp.float32)
        x_scr[0:N_SAMPLE, :] = St[0:N_SAMPLE, :]
        x_scr[N_SAMPLE:LTOT, :] = t_ref[0]
        x = x_scr[...]
        qp = jnp.dot(x, wq_ref[...], preferred_element_type=jnp.float32) + bq_ref[...]
        kp = jnp.dot(x, wk_ref[...], preferred_element_type=jnp.float32) + bk_ref[...]
        vp = jnp.dot(x, wv_ref[...], preferred_element_type=jnp.float32) + bv_ref[...]
        scale = 1.0 / math.sqrt(DH)
        o_heads = []
        for h in range(HEADS):
            c0 = h * DH
            qh = qp[:, c0:c0 + DH]
            kh = kp[:, c0:c0 + DH]
            vh = vp[:, c0:c0 + DH]
            s = lax.dot_general(qh, kh, (((1,), (1,)), ((), ())),
                                preferred_element_type=jnp.float32) * scale
            mx = jnp.max(s, axis=1, keepdims=True)
            e = jnp.exp(s - mx)
            a = e / jnp.sum(e, axis=1, keepdims=True)
            o_heads.append(jnp.dot(a, vh, preferred_element_type=jnp.float32))
        o = jnp.concatenate(o_heads, axis=1)
        out = jnp.dot(o, wo_ref[...], preferred_element_type=jnp.float32) + bo_ref[...]
        out_ref[0] = out


def _fused(xyz, masks, pf, t_feat, Wq, bq, Wk, bk, Wv, bv, Wo, bo,
           *, interpret=False):
    last = BATCH - 1
    return pl.pallas_call(
        _fused_body,
        grid=(BATCH + 1,),
        in_specs=[
            pl.BlockSpec((1, 3, NPTS), lambda i: (jnp.minimum(i, last), 0, 0)),
            pl.BlockSpec((1, NVIEW, NPTS),
                         lambda i: (jnp.minimum(i, last), 0, 0)),
            pl.BlockSpec(memory_space=pl.ANY),
            pl.BlockSpec((1, TTOK, EMB),
                         lambda i: (jnp.maximum(i - 1, 0), 0, 0)),
            pl.BlockSpec((EMB, EMB), lambda i: (0, 0)),
            pl.BlockSpec((1, EMB), lambda i: (0, 0)),
            pl.BlockSpec((EMB, EMB), lambda i: (0, 0)),
            pl.BlockSpec((1, EMB), lambda i: (0, 0)),
            pl.BlockSpec((EMB, EMB), lambda i: (0, 0)),
            pl.BlockSpec((1, EMB), lambda i: (0, 0)),
            pl.BlockSpec((EMB, EMB), lambda i: (0, 0)),
            pl.BlockSpec((1, EMB), lambda i: (0, 0)),
        ],
        out_specs=pl.BlockSpec((1, LTOT, EMB),
                               lambda i: (jnp.maximum(i - 1, 0), 0, 0)),
        out_shape=jax.ShapeDtypeStruct((BATCH, LTOT, EMB), jnp.float32),
        scratch_shapes=[
            pltpu.VMEM((LTOT, EMB), jnp.float32),
            pltpu.VMEM((2, EMB, N_SAMPLE * 128), jnp.float32),
            pltpu.VMEM((2, NVIEW, 16), jnp.int32),
            pltpu.SemaphoreType.DMA((2,)),
        ],
        interpret=interpret,
    )(xyz, masks, pf, t_feat, Wq, bq, Wk, bk, Wv, bv, Wo, bo)


def kernel(point_features, point_masks, t_feat, t_mask, xyz, Wq, bq, Wk, bk,
           Wv, bv, Wo, bo):
    output = _fused(xyz, point_masks, point_features, t_feat,
                    Wq, bq.reshape(1, EMB), Wk, bk.reshape(1, EMB),
                    Wv, bv.reshape(1, EMB), Wo, bo.reshape(1, EMB))
    combined_mask = jnp.concatenate(
        [jnp.ones((BATCH, N_SAMPLE), dtype=bool), t_mask], axis=1)
    return (output, combined_mask)


# speedup vs baseline: 1.4286x; 1.0544x over previous
import math

import jax
import jax.numpy as jnp
from jax import lax
from jax.experimental import pallas as pl
from jax.experimental.pallas import tpu as pltpu

N_SAMPLE = 20
EMB = 512
HEADS = 4
DH = EMB // HEADS
BATCH = 8
NPTS = 16384
TTOK = 64
NVIEW = 4
KPV = N_SAMPLE // NVIEW
LTOT = N_SAMPLE + TTOK
_CHUNKS = 16


def _tree(parts, fn):
    while len(parts) > 1:
        parts = [fn(parts[j], parts[j + 1]) if j + 1 < len(parts)
                 else parts[j] for j in range(0, len(parts), 2)]
    return parts[0]


def _rmin(x):
    w = x.shape[1] // _CHUNKS
    parts = [jnp.min(x[:, c * w:(c + 1) * w], axis=1, keepdims=True)
             for c in range(_CHUNKS)]
    return _tree(parts, jnp.minimum)


def _rsum(x):
    w = x.shape[1] // _CHUNKS
    parts = [jnp.sum(x[:, c * w:(c + 1) * w], axis=1, keepdims=True)
             for c in range(_CHUNKS)]
    return _tree(parts, jnp.add)


def _fused_body(xyz_ref, mask_ref, pf_ref, t_ref, wq_ref, bq_ref, wk_ref,
                bk_ref, wv_ref, bv_ref, wo_ref, bo_ref, out_ref,
                x_scr, blk_scr, idx_scr, sem):
    i = pl.program_id(0)
    p = lax.rem(i, 2)
    q = 1 - p
    col16 = lax.broadcasted_iota(jnp.int32, (NVIEW, 16), 1)
    row16 = lax.broadcasted_iota(jnp.int32, (NVIEW, 16), 0)
    inf = jnp.float32(jnp.inf)

    @pl.when(i < BATCH)
    def _compute():
        x3 = xyz_ref[0]
        m = mask_ref[0]
        cnt = jnp.clip(_rsum(m), 1.0, None)
        dist2 = jnp.zeros((NVIEW, NPTS), jnp.float32)
        for d in range(3):
            xd = x3[d:d + 1, :]
            cd = _rsum(m * xd) / cnt
            t = xd - cd
            dist2 = dist2 + t * t
        lane = lax.broadcasted_iota(jnp.int32, (NVIEW, NPTS), 1)
        arr = jnp.zeros((NVIEW, 16), jnp.int32)
        d2 = dist2
        for k in range(KPV):
            mn = _rmin(d2)
            cand = jnp.where(d2 == mn, lane, NPTS)
            mi = _rmin(cand)
            arr = jnp.where(col16 == k, mi, arr)
            d2 = jnp.where(lane == mi, inf, d2)
        idx_scr[p] = arr
        for v in range(NVIEW):
            for k in range(KPV):
                gF = jnp.sum(jnp.where((row16 == v) & (col16 == k), arr, 0))
                r = v * KPV + k
                pltpu.make_async_copy(
                    pf_ref.at[i, :, pl.ds(pl.multiple_of((gF // 128) * 128, 128), 128)],
                    blk_scr.at[p, :, pl.ds(r * 128, 128)],
                    sem.at[p],
                ).start()

    @pl.when(i > 0)
    def _process():
        for r in range(N_SAMPLE):
            pltpu.make_async_copy(
                pf_ref.at[0, :, pl.ds(0, 128)],
                blk_scr.at[q, :, pl.ds(r * 128, 128)],
                sem.at[q],
            ).wait()
        arr = idx_scr[q]
        lane128 = lax.broadcasted_iota(jnp.int32, (EMB, 128), 1)
        terms = []
        for r in range(N_SAMPLE):
            v_, k_ = r // KPV, r % KPV
            col = jnp.sum(jnp.where((row16 == v_) & (col16 == k_),
                                    arr, 0)) % 128
            blk = blk_scr[q, :, r * 128:(r + 1) * 128]
            rolled = pltpu.roll(blk, jnp.remainder(r - col, 128), axis=1)
            terms.append(jnp.where(lane128 == r, rolled, 0.0))
        while len(terms) > 1:
            terms = [terms[j] + terms[j + 1] if j + 1 < len(terms)
                     else terms[j] for j in range(0, len(terms), 2)]
        S = terms[0]
        eye = (lax.broadcasted_iota(jnp.int32, (EMB, EMB), 0)
               == lax.broadcasted_iota(jnp.int32, (EMB, EMB), 1)
               ).astype(jnp.float32)
        St = lax.dot_general(S, eye, (((0,), (0,)), ((), ())),
                             preferred_element_type=jnp.float32)
        x_scr[0:N_SAMPLE, :] = St[0:N_SAMPLE, :]
        x_scr[N_SAMPLE:LTOT, :] = t_ref[0]
        x = x_scr[...]
        qp = jnp.dot(x, wq_ref[...], preferred_element_type=jnp.float32) + bq_ref[...]
        kp = jnp.dot(x, wk_ref[...], preferred_element_type=jnp.float32) + bk_ref[...]
        vp = jnp.dot(x, wv_ref[...], preferred_element_type=jnp.float32) + bv_ref[...]
        scale = 1.0 / math.sqrt(DH)
        o_heads = []
        for h in range(HEADS):
            c0 = h * DH
            qh = qp[:, c0:c0 + DH]
            kh = kp[:, c0:c0 + DH]
            vh = vp[:, c0:c0 + DH]
            s = lax.dot_general(qh, kh, (((1,), (1,)), ((), ())),
                                preferred_element_type=jnp.float32) * scale
            mx = jnp.max(s, axis=1, keepdims=True)
            e = jnp.exp(s - mx)
            a = e / jnp.sum(e, axis=1, keepdims=True)
            o_heads.append(jnp.dot(a, vh, preferred_element_type=jnp.float32))
        o = jnp.concatenate(o_heads, axis=1)
        out = jnp.dot(o, wo_ref[...], preferred_element_type=jnp.float32) + bo_ref[...]
        out_ref[0] = out


def _fused(xyz, masks, pf, t_feat, Wq, bq, Wk, bk, Wv, bv, Wo, bo,
           *, interpret=False):
    last = BATCH - 1
    return pl.pallas_call(
        _fused_body,
        grid=(BATCH + 1,),
        in_specs=[
            pl.BlockSpec((1, 3, NPTS), lambda i: (jnp.minimum(i, last), 0, 0)),
            pl.BlockSpec((1, NVIEW, NPTS),
                         lambda i: (jnp.minimum(i, last), 0, 0)),
            pl.BlockSpec(memory_space=pl.ANY),
            pl.BlockSpec((1, TTOK, EMB),
                         lambda i: (jnp.maximum(i - 1, 0), 0, 0)),
            pl.BlockSpec((EMB, EMB), lambda i: (0, 0)),
            pl.BlockSpec((1, EMB), lambda i: (0, 0)),
            pl.BlockSpec((EMB, EMB), lambda i: (0, 0)),
            pl.BlockSpec((1, EMB), lambda i: (0, 0)),
            pl.BlockSpec((EMB, EMB), lambda i: (0, 0)),
            pl.BlockSpec((1, EMB), lambda i: (0, 0)),
            pl.BlockSpec((EMB, EMB), lambda i: (0, 0)),
            pl.BlockSpec((1, EMB), lambda i: (0, 0)),
        ],
        out_specs=pl.BlockSpec((1, LTOT, EMB),
                               lambda i: (jnp.maximum(i - 1, 0), 0, 0)),
        out_shape=jax.ShapeDtypeStruct((BATCH, LTOT, EMB), jnp.float32),
        scratch_shapes=[
            pltpu.VMEM((LTOT, EMB), jnp.float32),
            pltpu.VMEM((2, EMB, N_SAMPLE * 128), jnp.float32),
            pltpu.VMEM((2, NVIEW, 16), jnp.int32),
            pltpu.SemaphoreType.DMA((2,)),
        ],
        interpret=interpret,
    )(xyz, masks, pf, t_feat, Wq, bq, Wk, bk, Wv, bv, Wo, bo)


def kernel(point_features, point_masks, t_feat, t_mask, xyz, Wq, bq, Wk, bk,
           Wv, bv, Wo, bo):
    output = _fused(xyz, point_masks, point_features, t_feat,
                    Wq, bq.reshape(1, EMB), Wk, bk.reshape(1, EMB),
                    Wv, bv.reshape(1, EMB), Wo, bo.reshape(1, EMB))
    combined_mask = jnp.concatenate(
        [jnp.ones((BATCH, N_SAMPLE), dtype=bool), t_mask], axis=1)
    return (output, combined_mask)
